# baseline (speedup 1.0000x reference)
import functools

import jax
import jax.numpy as jnp
from jax import lax
from jax.experimental import pallas as pl
from jax.experimental.pallas import tpu as pltpu
from jax.experimental.pallas import tpu_sc as plsc

_NC = 2
_NS = 16
_NW = _NC * _NS

_OUT_DIM = 64
_HIST = 50
_BLK = 128
_SPW = 4


def _make_gather(batch: int):
    assert batch == _NW * _SPW * _BLK

    mesh = plsc.VectorSubcoreMesh(core_axis_name="c", subcore_axis_name="s")
    scratch = [pltpu.VMEM((_HIST, _SPW, _BLK), jnp.int32)]
    scratch += [pltpu.VMEM((_BLK, _OUT_DIM), jnp.float32)] * _SPW
    scratch += [pltpu.VMEM((_OUT_DIM, _BLK), jnp.float32)] * _SPW
    scratch += [pltpu.SemaphoreType.DMA] * (2 * _SPW)

    @functools.partial(
        pl.kernel,
        out_type=jax.ShapeDtypeStruct(
            (_HIST, 8, _BLK, 8, _BLK), jnp.float32
        ),
        mesh=mesh,
        scratch_types=scratch,
        compiler_params=pltpu.CompilerParams(use_tc_tiling_on_sc=False, needs_layout_passes=False),
    )
    def gather(table_hbm, idx_hbm, out_hbm, idx_v, *bufs):
        rows = bufs[:_SPW]
        tbuf = bufs[_SPW : 2 * _SPW]
        gsem = bufs[2 * _SPW : 3 * _SPW]
        ssem = bufs[3 * _SPW :]
        wid = lax.axis_index("s") * _NC + lax.axis_index("c")
        w4 = wid * _SPW
        pltpu.sync_copy(idx_hbm.at[:, pl.ds(w4, _SPW)], idx_v)

        iota = lax.iota(jnp.int32, 16)
        rowids = [iota + 16 * k for k in range(4)]

        def gather_copy(h, s):
            return pltpu.make_async_copy(
                table_hbm.at[idx_v.at[h, s]], rows[s], gsem[s]
            )

        def store_copy(h, s, a):
            return pltpu.make_async_copy(
                tbuf[s].at[pl.ds(8 * a, 8)],
                out_hbm.at[h, a, w4 + s],
                ssem[s],
            )

        zeros16 = iota * 0

        def transpose(s):
            rv, tv = rows[s], tbuf[s]

            def trow(r, cols):
                for k in range(4):
                    x = rv[r, pl.ds(16 * k, 16)]
                    plsc.store_scatter(tv, [rowids[k], cols], x)
                return cols + 1

            lax.fori_loop(0, _BLK, trow, zeros16)

        for s in range(_SPW):
            gather_copy(0, s).start()

        def hstep(h, carry):
            for s in range(_SPW):
                gather_copy(h, s).wait()

                @pl.when(h > 0)
                def _():
                    for a in range(8):
                        store_copy(h - 1, s, a).wait()

                transpose(s)

                @pl.when(h < _HIST - 1)
                def _():
                    gather_copy(h + 1, s).start()

                for a in range(8):
                    store_copy(h, s, a).start()
            return carry

        lax.fori_loop(0, _HIST, hstep, 0)

        for s in range(_SPW):
            for a in range(8):
                store_copy(_HIST - 1, s, a).wait()

    return gather


def kernel(input_ids, table):
    batch, hist = input_ids.shape
    idxT3 = input_ids.T.reshape(hist, batch // _BLK, _BLK).astype(jnp.int32)
    out5 = _make_gather(batch)(table, idxT3)
    return out5.transpose(2, 4, 0, 1, 3).reshape(batch, hist, _OUT_DIM)


# speedup vs baseline: 1.4160x; 1.4160x over previous
import functools

import jax
import jax.numpy as jnp
from jax import lax
from jax.experimental import pallas as pl
from jax.experimental.pallas import tpu as pltpu
from jax.experimental.pallas import tpu_sc as plsc

_NC = 2
_NS = 16
_NW = _NC * _NS

_OUT_DIM = 64
_HIST = 50
_CHUNK = 128
_K = 4


def _make_gather(n_rows: int):
    assert n_rows % (_NW * _CHUNK * 2 * _K) == 0
    chunks_per_w = n_rows // (_NW * _CHUNK)
    n_pairs = chunks_per_w // (2 * _K)

    mesh = plsc.VectorSubcoreMesh(core_axis_name="c", subcore_axis_name="s")
    nbuf = 2 * _K
    scratch = [pltpu.VMEM((chunks_per_w, _CHUNK), jnp.int32)]
    scratch += [pltpu.VMEM((_CHUNK, _OUT_DIM), jnp.float32)] * nbuf
    scratch += [pltpu.SemaphoreType.DMA] * (2 * nbuf)

    @functools.partial(
        pl.kernel,
        out_type=jax.ShapeDtypeStruct((n_rows, _OUT_DIM), jnp.float32),
        mesh=mesh,
        scratch_types=scratch,
        compiler_params=pltpu.CompilerParams(use_tc_tiling_on_sc=False),
    )
    def gather(table_hbm, idx_hbm, out_hbm, idx_v, *bufs):
        rows = bufs[:nbuf]
        gsem = bufs[nbuf : 2 * nbuf]
        ssem = bufs[2 * nbuf :]
        wid = lax.axis_index("s") * _NC + lax.axis_index("c")
        cbase = wid * chunks_per_w
        pltpu.sync_copy(idx_hbm.at[pl.ds(cbase, chunks_per_w)], idx_v)

        def gather_copy(j, b):
            return pltpu.make_async_copy(
                table_hbm.at[idx_v.at[j]], rows[b], gsem[b]
            )

        def store_copy(j, b):
            return pltpu.make_async_copy(
                rows[b], out_hbm.at[pl.ds((cbase + j) * _CHUNK, _CHUNK)], ssem[b]
            )

        for s in range(_K):
            gather_copy(s, s).start()

        def pair(t, carry):
            g0 = 2 * t
            for s in range(_K):
                b = _K + s

                @pl.when(t > 0)
                def _():
                    store_copy((g0 - 1) * _K + s, b).wait()

                gather_copy((g0 + 1) * _K + s, b).start()
            for s in range(_K):
                gather_copy(g0 * _K + s, s).wait()
                store_copy(g0 * _K + s, s).start()
            for s in range(_K):
                store_copy(g0 * _K + s, s).wait()

                @pl.when(t < n_pairs - 1)
                def _():
                    gather_copy((g0 + 2) * _K + s, s).start()
            for s in range(_K):
                b = _K + s
                gather_copy((g0 + 1) * _K + s, b).wait()
                store_copy((g0 + 1) * _K + s, b).start()
            return carry

        lax.fori_loop(0, n_pairs, pair, 0)

        for s in range(_K):
            store_copy((2 * n_pairs - 1) * _K + s, _K + s).wait()

    return gather


def _relayout_body(flat_ref, out_ref):
    x = flat_ref[...]
    y = jnp.swapaxes(x, 0, 1)
    out_ref[...] = y.reshape(_HIST, 8, 1, 8, _CHUNK)


def _make_relayout(batch: int):
    nblk = batch // _CHUNK
    return pl.pallas_call(
        _relayout_body,
        grid=(nblk,),
        in_specs=[
            pl.BlockSpec((_CHUNK, _HIST * _OUT_DIM), lambda i: (i, 0))
        ],
        out_specs=pl.BlockSpec(
            (_HIST, 8, 1, 8, _CHUNK), lambda i: (0, 0, i, 0, 0)
        ),
        out_shape=jax.ShapeDtypeStruct(
            (_HIST, 8, nblk, 8, _CHUNK), jnp.float32
        ),
    )


def kernel(input_ids, table):
    batch, hist = input_ids.shape
    n = batch * hist
    idx2d = input_ids.reshape(n // _CHUNK, _CHUNK).astype(jnp.int32)
    flat = _make_gather(n)(table, idx2d)
    flat2 = flat.reshape(batch, hist * _OUT_DIM)
    out5 = _make_relayout(batch)(flat2)
    return out5.transpose(2, 4, 0, 1, 3).reshape(batch, hist, _OUT_DIM)


# speedup vs baseline: 1.4162x; 1.0002x over previous
import functools

import jax
import jax.numpy as jnp
from jax import lax
from jax.experimental import pallas as pl
from jax.experimental.pallas import tpu as pltpu
from jax.experimental.pallas import tpu_sc as plsc

_NC = 2
_NS = 16
_NW = _NC * _NS

_OUT_DIM = 64
_PAD = 128
_HIST = 50
_BLK = 128
_SPW = 4
_SUB = 2048


def _make_gather(batch: int):
    assert batch == _NW * _SPW * _BLK

    mesh = plsc.VectorSubcoreMesh(core_axis_name="c", subcore_axis_name="s")
    scratch = [pltpu.VMEM((_HIST, _SPW, _BLK), jnp.int32)]
    scratch += [pltpu.VMEM((_BLK, _OUT_DIM), jnp.float32)] * _SPW
    scratch += [pltpu.SemaphoreType.DMA] * (2 * _SPW)

    @functools.partial(
        pl.kernel,
        out_type=jax.ShapeDtypeStruct((_HIST * batch, _PAD), jnp.float32),
        mesh=mesh,
        scratch_types=scratch,
        compiler_params=pltpu.CompilerParams(use_tc_tiling_on_sc=False),
    )
    def gather(table_hbm, idx_hbm, out_hbm, idx_v, *bufs):
        rows = bufs[:_SPW]
        gsem = bufs[_SPW : 2 * _SPW]
        ssem = bufs[2 * _SPW :]
        wid = lax.axis_index("s") * _NC + lax.axis_index("c")
        w4 = wid * _SPW
        pltpu.sync_copy(idx_hbm.at[:, pl.ds(w4, _SPW)], idx_v)

        def gather_copy(h, s):
            return pltpu.make_async_copy(
                table_hbm.at[idx_v.at[h, s]], rows[s], gsem[s]
            )

        def store_copy(h, s):
            base = h * batch + (w4 + s) * _BLK
            return pltpu.make_async_copy(
                rows[s],
                out_hbm.at[pl.ds(base, _BLK), pl.ds(0, _OUT_DIM)],
                ssem[s],
            )

        for s in range(_SPW):
            gather_copy(0, s).start()

        def hstep(h, carry):
            for s in range(_SPW):
                gather_copy(h, s).wait()

                @pl.when(h > 0)
                def _():
                    store_copy(h - 1, s).wait()

                @pl.when(h < _HIST - 1)
                def _():
                    gather_copy(h + 1, s).start()

                store_copy(h, s).start()
            return carry

        lax.fori_loop(0, _HIST, hstep, 0)

        for s in range(_SPW):
            store_copy(_HIST - 1, s).wait()

    return gather


def _relayout_body(flat_ref, out_ref):
    t = jnp.swapaxes(flat_ref[:, : _OUT_DIM], 0, 1)
    for a in range(8):
        for j in range(16):
            out_ref[0, a, j, :, :] = t[
                8 * a : 8 * a + 8, _BLK * j : _BLK * (j + 1)
            ]


def _make_relayout(batch: int):
    nblk = batch // _BLK
    nsub = batch // _SUB
    return pl.pallas_call(
        _relayout_body,
        grid=(_HIST, nsub),
        in_specs=[
            pl.BlockSpec((_SUB, _PAD), lambda h, u: (h * nsub + u, 0))
        ],
        out_specs=pl.BlockSpec(
            (1, 8, _SUB // _BLK, 8, _BLK), lambda h, u: (h, 0, u, 0, 0)
        ),
        out_shape=jax.ShapeDtypeStruct(
            (_HIST, 8, nblk, 8, _BLK), jnp.float32
        ),
    )


def kernel(input_ids, table):
    batch, hist = input_ids.shape
    idxT3 = input_ids.T.reshape(hist, batch // _BLK, _BLK).astype(jnp.int32)
    flat_pad = _make_gather(batch)(table, idxT3)
    out5 = _make_relayout(batch)(flat_pad)
    return out5.transpose(2, 4, 0, 1, 3).reshape(batch, hist, _OUT_DIM)


# speedup vs baseline: 1.4177x; 1.0010x over previous
import functools

import jax
import jax.numpy as jnp
from jax import lax
from jax.experimental import pallas as pl
from jax.experimental.pallas import tpu as pltpu
from jax.experimental.pallas import tpu_sc as plsc

_NC = 2
_NS = 16
_NW = _NC * _NS

_OUT_DIM = 64
_HIST = 50
_CHUNK = 128
_K = 4


def _make_gather(n_rows: int):
    assert n_rows % (_NW * _CHUNK * 2 * _K) == 0
    chunks_per_w = n_rows // (_NW * _CHUNK)
    n_pairs = chunks_per_w // (2 * _K)

    mesh = plsc.VectorSubcoreMesh(core_axis_name="c", subcore_axis_name="s")
    nbuf = 2 * _K
    scratch = [pltpu.VMEM((chunks_per_w, _CHUNK), jnp.int32)]
    scratch += [pltpu.VMEM((_CHUNK, _OUT_DIM), jnp.float32)] * nbuf
    scratch += [pltpu.SemaphoreType.DMA] * (2 * nbuf)

    @functools.partial(
        pl.kernel,
        out_type=jax.ShapeDtypeStruct((n_rows, _OUT_DIM), jnp.float32),
        mesh=mesh,
        scratch_types=scratch,
        compiler_params=pltpu.CompilerParams(use_tc_tiling_on_sc=False),
    )
    def gather(table_hbm, idx_hbm, out_hbm, idx_v, *bufs):
        rows = bufs[:nbuf]
        gsem = bufs[nbuf : 2 * nbuf]
        ssem = bufs[2 * nbuf :]
        wid = lax.axis_index("s") * _NC + lax.axis_index("c")
        cbase = wid * chunks_per_w
        pltpu.sync_copy(idx_hbm.at[pl.ds(cbase, chunks_per_w)], idx_v)

        def gather_copy(j, b):
            return pltpu.make_async_copy(
                table_hbm.at[idx_v.at[j]], rows[b], gsem[b]
            )

        def store_copy(j, b):
            return pltpu.make_async_copy(
                rows[b], out_hbm.at[pl.ds((cbase + j) * _CHUNK, _CHUNK)], ssem[b]
            )

        for s in range(_K):
            gather_copy(s, s).start()

        def pair(t, carry):
            g0 = 2 * t
            for s in range(_K):
                b = _K + s

                @pl.when(t > 0)
                def _():
                    store_copy((g0 - 1) * _K + s, b).wait()

                gather_copy((g0 + 1) * _K + s, b).start()
            for s in range(_K):
                gather_copy(g0 * _K + s, s).wait()
                store_copy(g0 * _K + s, s).start()
            for s in range(_K):
                store_copy(g0 * _K + s, s).wait()

                @pl.when(t < n_pairs - 1)
                def _():
                    gather_copy((g0 + 2) * _K + s, s).start()
            for s in range(_K):
                b = _K + s
                gather_copy((g0 + 1) * _K + s, b).wait()
                store_copy((g0 + 1) * _K + s, b).start()
            return carry

        lax.fori_loop(0, n_pairs, pair, 0)

        for s in range(_K):
            store_copy((2 * n_pairs - 1) * _K + s, _K + s).wait()

    return gather


def _relayout_body(flat_ref, out_ref):
    x = flat_ref[...]
    y = jnp.swapaxes(x, 0, 1)
    out_ref[...] = y.reshape(_HIST, 8, 1, 8, _CHUNK)


def _make_relayout(batch: int):
    nblk = batch // _CHUNK
    return pl.pallas_call(
        _relayout_body,
        grid=(nblk,),
        in_specs=[
            pl.BlockSpec((_CHUNK, _HIST * _OUT_DIM), lambda i: (i, 0))
        ],
        out_specs=pl.BlockSpec(
            (_HIST, 8, 1, 8, _CHUNK), lambda i: (0, 0, i, 0, 0)
        ),
        out_shape=jax.ShapeDtypeStruct(
            (_HIST, 8, nblk, 8, _CHUNK), jnp.float32
        ),
    )


def kernel(input_ids, table):
    batch, hist = input_ids.shape
    n = batch * hist
    idx2d = input_ids.reshape(n // _CHUNK, _CHUNK).astype(jnp.int32)
    flat = _make_gather(n)(table, idx2d)
    flat2 = flat.reshape(batch, hist * _OUT_DIM)
    out5 = _make_relayout(batch)(flat2)
    return out5.transpose(2, 4, 0, 1, 3).reshape(batch, hist, _OUT_DIM)
